# Initial kernel scaffold; baseline (speedup 1.0000x reference)
import jax
import jax.numpy as jnp
from jax.experimental import pallas as pl


def kernel(x, edge_index, batch, W0, b0, W1, b1, W2, b2, g0, be0, g1, be1, g2, be2, Wa, ba, Wb, bb):
    raise NotImplementedError("write your pallas kernel here")


# speedup vs baseline: 7.3402x; 7.3402x over previous
import functools

import jax
import jax.numpy as jnp
from jax import lax
from jax.experimental import pallas as pl
from jax.experimental.pallas import tpu as pltpu
from jax.experimental.pallas import tpu_sc as plsc

_N = 10000
_D = 128
_G = 16
_E = 320000

_NC = 2
_NS = 16
_NW = _NC * _NS
_B = 128
_KCH = 80
_EPW = _B * _KCH
_EPAD = _EPW * _NW
_NACC = 10240
_RPT = _NACC // _NS
_ZCH = _RPT // _B

_mesh = plsc.VectorSubcoreMesh(
    core_axis_name="c", subcore_axis_name="s", num_cores=_NC, num_subcores=_NS
)


def _fill(ref, rows, width, val):
    vecs = width // 16

    def body(i, carry):
        r = i // vecs
        col = (i % vecs) * 16
        ref[r, pl.ds(col, 16)] = jnp.full((16,), val, jnp.float32)
        return carry

    lax.fori_loop(0, rows * vecs, body, 0)


def _zero_fill(ref, rows, width):
    _fill(ref, rows, width, 0.0)


@functools.partial(
    pl.kernel,
    out_type=jax.ShapeDtypeStruct((_NC, _NACC, _D), jnp.float32),
    mesh=_mesh,
    scratch_types=[
        pltpu.VMEM((_KCH, _B), jnp.int32),
        pltpu.VMEM((_B, _D), jnp.float32),
        pltpu.VMEM_SHARED((_NACC, _D), jnp.float32),
    ],
)
def _deg_sc(dst_hbm, out_hbm, dst_v, buf, acc_s):
    c = lax.axis_index("c")
    s = lax.axis_index("s")
    wid = s * _NC + c

    pltpu.sync_copy(dst_hbm.at[wid], dst_v)

    _zero_fill(buf, _B, _D)
    for k in range(_ZCH):
        pltpu.sync_copy(buf, acc_s.at[pl.ds(s * _RPT + k * _B, _B)])
    plsc.subcore_barrier()

    _fill(buf, _B, _D, 1.0)

    def body(j, carry):
        pltpu.sync_copy(buf, acc_s.at[dst_v.at[j]], add=True)
        return carry

    lax.fori_loop(0, _KCH, body, 0)
    plsc.subcore_barrier()

    base = s * _RPT
    for k in range(_ZCH):
        r0 = base + k * _B
        pltpu.sync_copy(acc_s.at[pl.ds(r0, _B)], buf)
        pltpu.sync_copy(buf, out_hbm.at[c, pl.ds(r0, _B)])


@functools.partial(
    pl.kernel,
    out_type=jax.ShapeDtypeStruct((_NC, _NACC, _D), jnp.float32),
    mesh=_mesh,
    scratch_types=[
        pltpu.VMEM((_KCH, _B), jnp.int32),
        pltpu.VMEM((2, _B), jnp.int32),
        pltpu.VMEM((2, _B), jnp.int32),
        pltpu.VMEM((_B, _D), jnp.float32),
        pltpu.VMEM((_B, _D), jnp.float32),
        pltpu.VMEM_SHARED((_NACC, _D), jnp.float32),
        pltpu.SemaphoreType.DMA,
        pltpu.SemaphoreType.DMA,
    ],
)
def _spmm_sc(hd_hbm, pk_hbm, out_hbm, pk_v, src_c, dst_c, rb0, rb1, acc_s,
             sem0, sem1):
    c = lax.axis_index("c")
    s = lax.axis_index("s")
    wid = s * _NC + c

    pltpu.sync_copy(pk_hbm.at[wid], pk_v)

    def unpack(j, slot):
        def go(v, carry):
            pk = pk_v[j, pl.ds(v * 16, 16)]
            src_c[slot, pl.ds(v * 16, 16)] = jnp.bitwise_and(pk, 16383)
            dst_c[slot, pl.ds(v * 16, 16)] = lax.shift_right_logical(pk, 14)
            return carry

        lax.fori_loop(0, _B // 16, go, 0)

    _zero_fill(rb0, _B, _D)
    for k in range(_ZCH):
        pltpu.sync_copy(rb0, acc_s.at[pl.ds(s * _RPT + k * _B, _B)])
    plsc.subcore_barrier()

    unpack(0, 0)
    pltpu.async_copy(hd_hbm.at[src_c.at[0]], rb0, sem0)

    def body(g, carry):
        j1 = 2 * g + 1
        unpack(j1, 1)
        pltpu.make_async_copy(hd_hbm.at[src_c.at[0]], rb0, sem0).wait()
        pltpu.async_copy(hd_hbm.at[src_c.at[1]], rb1, sem1)
        pltpu.sync_copy(rb0, acc_s.at[dst_c.at[0]], add=True)
        jn = lax.rem(2 * g + 2, _KCH)
        unpack(jn, 0)
        pltpu.async_copy(hd_hbm.at[src_c.at[0]], rb0, sem0)
        pltpu.make_async_copy(hd_hbm.at[src_c.at[1]], rb1, sem1).wait()
        pltpu.sync_copy(rb1, acc_s.at[dst_c.at[1]], add=True)
        return carry

    lax.fori_loop(0, _KCH // 2, body, 0)
    pltpu.make_async_copy(hd_hbm.at[src_c.at[0]], rb0, sem0).wait()
    plsc.subcore_barrier()

    base = s * _RPT
    for k in range(_ZCH):
        r0 = base + k * _B
        pltpu.sync_copy(acc_s.at[pl.ds(r0, _B)], rb0)
        pltpu.sync_copy(rb0, out_hbm.at[c, pl.ds(r0, _B)])


def _pre_body(degp_ref, x_ref, w_ref, hd_ref, dinv_ref):
    d = degp_ref[...]
    deg = 1.0 + d[0, :_N, 0] + d[1, :_N, 0]
    dinv = lax.rsqrt(deg)[:, None]
    dinvb = jnp.broadcast_to(dinv, (_N, _D))
    h = jnp.dot(x_ref[...], w_ref[...], preferred_element_type=jnp.float32)
    hd_ref[...] = h * dinvb
    dinv_ref[...] = dinvb


_pre_tc = pl.pallas_call(
    _pre_body,
    out_shape=[
        jax.ShapeDtypeStruct((_N, _D), jnp.float32),
        jax.ShapeDtypeStruct((_N, _D), jnp.float32),
    ],
)


def _bn_relu(t, gamma, beta):
    m = jnp.mean(t, axis=0)
    v = jnp.mean((t - m[None, :]) ** 2, axis=0)
    t = (t - m[None, :]) * lax.rsqrt(v + 1e-5) * gamma[None, :] + beta[None, :]
    return jnp.maximum(t, 0.0)


def _mid_body(accp_ref, hd_ref, dinv_ref, b_ref, g_ref, be_ref, w_ref, o_ref):
    a = accp_ref[...]
    dinvb = dinv_ref[...]
    t = (a[0, :_N] + a[1, :_N] + hd_ref[...]) * dinvb + b_ref[...][None, :]
    t = _bn_relu(t, g_ref[...], be_ref[...])
    o_ref[...] = jnp.dot(t, w_ref[...], preferred_element_type=jnp.float32) * dinvb


_mid_tc = pl.pallas_call(
    _mid_body,
    out_shape=jax.ShapeDtypeStruct((_N, _D), jnp.float32),
)


def _fin_body(accp_ref, hd_ref, dinv_ref, b_ref, g_ref, be_ref, batch_ref,
              wa_ref, ba_ref, wb_ref, bb_ref, o_ref):
    a = accp_ref[...]
    t = (a[0, :_N] + a[1, :_N] + hd_ref[...]) * dinv_ref[...] + b_ref[...][None, :]
    t = _bn_relu(t, g_ref[...], be_ref[...])
    bvec = batch_ref[...]
    gids = lax.broadcasted_iota(jnp.int32, (_N, _G), 1)
    onehot = (bvec == gids).astype(jnp.float32)
    cnt = jnp.sum(onehot, axis=0)
    pooled_sum = jax.lax.dot_general(
        onehot, t, (((0,), (0,)), ((), ())),
        preferred_element_type=jnp.float32)
    pooled = pooled_sum / jnp.maximum(cnt, 1.0)[:, None]
    z = jnp.maximum(
        jnp.dot(pooled, wa_ref[...], preferred_element_type=jnp.float32)
        + ba_ref[...][None, :], 0.0)
    o_ref[...] = (jnp.dot(z, wb_ref[...], preferred_element_type=jnp.float32)
                  + bb_ref[...][None, :])


_fin_tc = pl.pallas_call(
    _fin_body,
    out_shape=jax.ShapeDtypeStruct((_G, 1), jnp.float32),
)


def kernel(x, edge_index, batch, W0, b0, W1, b1, W2, b2,
           g0, be0, g1, be1, g2, be2, Wa, ba, Wb, bb):
    src = edge_index[0].astype(jnp.int32)
    dst = edge_index[1].astype(jnp.int32)
    pad = _EPAD - _E
    dstp = jnp.concatenate([dst, jnp.full((pad,), _N, jnp.int32)]).reshape(
        _NW, _KCH, _B)
    pk = (jnp.concatenate([dst * 16384 + src,
                           jnp.full((pad,), _N * 16384, jnp.int32)])
          .reshape(_NW, _KCH, _B))

    degp = _deg_sc(dstp)
    hd, dinvb = _pre_tc(degp, x, W0)
    acc = _spmm_sc(hd, pk)
    hd = _mid_tc(acc, hd, dinvb, b0, g0, be0, W1)
    acc = _spmm_sc(hd, pk)
    hd = _mid_tc(acc, hd, dinvb, b1, g1, be1, W2)
    acc = _spmm_sc(hd, pk)
    out = _fin_tc(acc, hd, dinvb, b2, g2, be2,
                  batch.astype(jnp.int32).reshape(_N, 1), Wa, ba, Wb, bb)
    return out


# speedup vs baseline: 9.0505x; 1.2330x over previous
import functools

import jax
import jax.numpy as jnp
from jax import lax
from jax.experimental import pallas as pl
from jax.experimental.pallas import tpu as pltpu
from jax.experimental.pallas import tpu_sc as plsc

_N = 10000
_D = 128
_G = 16
_E = 320000

_NC = 2
_NS = 16
_NW = _NC * _NS
_B = 128
_KCH = 80
_EPW = _B * _KCH
_EPAD = _EPW * _NW
_NACC = 10240
_RPT = _NACC // _NS
_ZCH = _RPT // _B

_mesh = plsc.VectorSubcoreMesh(
    core_axis_name="c", subcore_axis_name="s", num_cores=_NC, num_subcores=_NS
)


def _fill(ref, rows, width, val):
    vecs = width // 16

    def body(i, carry):
        r = i // vecs
        col = (i % vecs) * 16
        ref[r, pl.ds(col, 16)] = jnp.full((16,), val, jnp.float32)
        return carry

    lax.fori_loop(0, rows * vecs, body, 0)


def _zero_fill(ref, rows, width):
    _fill(ref, rows, width, 0.0)


@functools.partial(
    pl.kernel,
    out_type=jax.ShapeDtypeStruct((_NC, _NACC, _D), jnp.float32),
    mesh=_mesh,
    scratch_types=[
        pltpu.VMEM((_KCH, _B), jnp.int32),
        pltpu.VMEM((_B, _D), jnp.float32),
        pltpu.VMEM_SHARED((_NACC, _D), jnp.float32),
    ],
)
def _deg_sc(dst_hbm, out_hbm, dst_v, buf, acc_s):
    c = lax.axis_index("c")
    s = lax.axis_index("s")
    wid = s * _NC + c

    pltpu.sync_copy(dst_hbm.at[wid], dst_v)

    _zero_fill(buf, _B, _D)
    for k in range(_ZCH):
        pltpu.sync_copy(buf, acc_s.at[pl.ds(s * _RPT + k * _B, _B)])
    plsc.subcore_barrier()

    _fill(buf, _B, _D, 1.0)

    def body(j, carry):
        pltpu.sync_copy(buf, acc_s.at[dst_v.at[j]], add=True)
        return carry

    lax.fori_loop(0, _KCH, body, 0)
    plsc.subcore_barrier()

    base = s * _RPT
    for k in range(_ZCH):
        r0 = base + k * _B
        pltpu.sync_copy(acc_s.at[pl.ds(r0, _B)], buf)
        pltpu.sync_copy(buf, out_hbm.at[c, pl.ds(r0, _B)])


@functools.partial(
    pl.kernel,
    out_type=jax.ShapeDtypeStruct((_NC, _NACC, _D), jnp.float32),
    mesh=_mesh,
    scratch_types=[
        pltpu.VMEM((_KCH, _B), jnp.int32),
        pltpu.VMEM((2, _B), jnp.int32),
        pltpu.VMEM((2, _B), jnp.int32),
        pltpu.VMEM((_B, _D), jnp.float32),
        pltpu.VMEM((_B, _D), jnp.float32),
        pltpu.VMEM_SHARED((_NACC, _D), jnp.float32),
        pltpu.SemaphoreType.DMA,
        pltpu.SemaphoreType.DMA,
    ],
)
def _spmm_sc(hd_hbm, pk_hbm, out_hbm, pk_v, src_c, dst_c, rb0, rb1, acc_s,
             sem0, sem1):
    c = lax.axis_index("c")
    s = lax.axis_index("s")
    wid = s * _NC + c

    pltpu.sync_copy(pk_hbm.at[wid], pk_v)

    def unpack(j, slot):
        def go(v, carry):
            pk = pk_v[j, pl.ds(v * 16, 16)]
            src_c[slot, pl.ds(v * 16, 16)] = jnp.bitwise_and(pk, 16383)
            dst_c[slot, pl.ds(v * 16, 16)] = lax.shift_right_logical(pk, 14)
            return carry

        lax.fori_loop(0, _B // 16, go, 0)

    _zero_fill(rb0, _B, _D)
    for k in range(_ZCH):
        pltpu.sync_copy(rb0, acc_s.at[pl.ds(s * _RPT + k * _B, _B)])
    plsc.subcore_barrier()

    unpack(0, 0)
    pltpu.async_copy(hd_hbm.at[src_c.at[0]], rb0, sem0)

    def body(g, carry):
        j1 = 2 * g + 1
        unpack(j1, 1)
        pltpu.make_async_copy(hd_hbm.at[src_c.at[0]], rb0, sem0).wait()
        pltpu.async_copy(hd_hbm.at[src_c.at[1]], rb1, sem1)
        pltpu.sync_copy(rb0, acc_s.at[dst_c.at[0]], add=True)
        jn = lax.rem(2 * g + 2, _KCH)
        unpack(jn, 0)
        pltpu.async_copy(hd_hbm.at[src_c.at[0]], rb0, sem0)
        pltpu.make_async_copy(hd_hbm.at[src_c.at[1]], rb1, sem1).wait()
        pltpu.sync_copy(rb1, acc_s.at[dst_c.at[1]], add=True)
        return carry

    lax.fori_loop(0, _KCH // 2, body, 0)
    pltpu.make_async_copy(hd_hbm.at[src_c.at[0]], rb0, sem0).wait()
    plsc.subcore_barrier()

    base = s * _RPT
    for k in range(_ZCH):
        r0 = base + k * _B
        pltpu.sync_copy(acc_s.at[pl.ds(r0, _B)], rb0)
        pltpu.sync_copy(rb0, out_hbm.at[c, pl.ds(r0, _B)])


def _pre_body(degp_ref, x_ref, w_ref, hd_ref, dinv_ref):
    d = degp_ref[...]
    deg = 1.0 + d[0, :_N, 0] + d[1, :_N, 0]
    dinv = lax.rsqrt(deg)[:, None]
    dinvb = jnp.broadcast_to(dinv, (_N, _D))
    h = jnp.dot(x_ref[...], w_ref[...], preferred_element_type=jnp.float32)
    hd_ref[...] = h * dinvb
    dinv_ref[...] = dinvb


_pre_tc = pl.pallas_call(
    _pre_body,
    out_shape=[
        jax.ShapeDtypeStruct((_N, _D), jnp.float32),
        jax.ShapeDtypeStruct((_N, _D), jnp.float32),
    ],
)


def _bn_relu(t, gamma, beta):
    m = jnp.mean(t, axis=0)
    v = jnp.mean((t - m[None, :]) ** 2, axis=0)
    t = (t - m[None, :]) * lax.rsqrt(v + 1e-5) * gamma[None, :] + beta[None, :]
    return jnp.maximum(t, 0.0)


def _mid_body(accp_ref, hd_ref, dinv_ref, b_ref, g_ref, be_ref, w_ref, o_ref):
    a = accp_ref[...]
    dinvb = dinv_ref[...]
    t = (a[0, :_N] + a[1, :_N] + hd_ref[...]) * dinvb + b_ref[...][None, :]
    t = _bn_relu(t, g_ref[...], be_ref[...])
    o_ref[...] = jnp.dot(t, w_ref[...], preferred_element_type=jnp.float32) * dinvb


_mid_tc = pl.pallas_call(
    _mid_body,
    out_shape=jax.ShapeDtypeStruct((_N, _D), jnp.float32),
)


def _fin_body(accp_ref, hd_ref, dinv_ref, b_ref, g_ref, be_ref, batch_ref,
              wa_ref, ba_ref, wb_ref, bb_ref, o_ref):
    a = accp_ref[...]
    t = (a[0, :_N] + a[1, :_N] + hd_ref[...]) * dinv_ref[...] + b_ref[...][None, :]
    t = _bn_relu(t, g_ref[...], be_ref[...])
    bvec = batch_ref[...]
    gids = lax.broadcasted_iota(jnp.int32, (_N, _G), 1)
    onehot = (bvec == gids).astype(jnp.float32)
    cnt = jnp.sum(onehot, axis=0)
    pooled_sum = jax.lax.dot_general(
        onehot, t, (((0,), (0,)), ((), ())),
        preferred_element_type=jnp.float32)
    pooled = pooled_sum / jnp.maximum(cnt, 1.0)[:, None]
    z = jnp.maximum(
        jnp.dot(pooled, wa_ref[...], preferred_element_type=jnp.float32)
        + ba_ref[...][None, :], 0.0)
    o_ref[...] = (jnp.dot(z, wb_ref[...], preferred_element_type=jnp.float32)
                  + bb_ref[...][None, :])


_fin_tc = pl.pallas_call(
    _fin_body,
    out_shape=jax.ShapeDtypeStruct((_G, 1), jnp.float32),
)


def kernel(x, edge_index, batch, W0, b0, W1, b1, W2, b2,
           g0, be0, g1, be1, g2, be2, Wa, ba, Wb, bb):
    src = edge_index[0].astype(jnp.int32)
    dst = edge_index[1].astype(jnp.int32)
    ppw = _EPW - _E // _NW
    dstp = jnp.concatenate(
        [dst.reshape(_NW, _E // _NW),
         jnp.full((_NW, ppw), _N, jnp.int32)], axis=1).reshape(
        _NW, _KCH, _B)
    pk = jnp.concatenate(
        [(dst * 16384 + src).reshape(_NW, _E // _NW),
         jnp.full((_NW, ppw), _N * 16384, jnp.int32)], axis=1).reshape(
        _NW, _KCH, _B)

    degp = _deg_sc(dstp)
    hd, dinvb = _pre_tc(degp, x, W0)
    acc = _spmm_sc(hd, pk)
    hd = _mid_tc(acc, hd, dinvb, b0, g0, be0, W1)
    acc = _spmm_sc(hd, pk)
    hd = _mid_tc(acc, hd, dinvb, b1, g1, be1, W2)
    acc = _spmm_sc(hd, pk)
    out = _fin_tc(acc, hd, dinvb, b2, g2, be2,
                  batch.astype(jnp.int32).reshape(_N, 1), Wa, ba, Wb, bb)
    return out
